# baseline (speedup 1.0000x reference)
import functools

import jax
import jax.numpy as jnp
from jax import lax
from jax.experimental import pallas as pl
from jax.experimental.pallas import tpu as pltpu
from jax.experimental.pallas import tpu_sc as plsc

MAXLEN = 200
EMB = 64
BATCH = 4096

NC = 2
NS = 16
NW = NC * NS

B_PER_W = BATCH // NW
HALF = 100


def _emb_body(idx_hbm, tok_hbm, pos_hbm, out_hbm, pos_v, idx_v, rows_v, gsem):
    wid = lax.axis_index("s") * NC + lax.axis_index("c")
    pltpu.sync_copy(pos_hbm, pos_v)

    def chunk_body(g, carry):
        belem = wid * B_PER_W + g
        row0 = belem * MAXLEN
        pltpu.sync_copy(idx_hbm.at[pl.ds(belem * 2, 2)], idx_v)
        cp0 = pltpu.async_copy(tok_hbm.at[idx_v.at[0]],
                               rows_v.at[pl.ds(0, HALF)], gsem)
        cp1 = pltpu.async_copy(tok_hbm.at[idx_v.at[1]],
                               rows_v.at[pl.ds(HALF, HALF)], gsem)
        cp0.wait()
        cp1.wait()

        def add_body(r, c):
            for cc in range(EMB // 16):
                sl = pl.ds(cc * 16, 16)
                rows_v[r, sl] = rows_v[r, sl] + pos_v[r, sl]
            return c

        lax.fori_loop(0, MAXLEN, add_body, 0)
        pltpu.sync_copy(rows_v, out_hbm.at[pl.ds(row0, MAXLEN)])
        return carry

    lax.fori_loop(0, B_PER_W, chunk_body, 0)


@jax.jit
def kernel(x, token_table, pos_table):
    idx = x.astype(jnp.int32).reshape(BATCH * MAXLEN // HALF, HALF)
    mesh = plsc.VectorSubcoreMesh(core_axis_name="c", subcore_axis_name="s")
    out = pl.kernel(
        _emb_body,
        mesh=mesh,
        compiler_params=pltpu.CompilerParams(use_tc_tiling_on_sc=False),
        out_type=jax.ShapeDtypeStruct((BATCH * MAXLEN, EMB), jnp.float32),
        scratch_types=[
            pltpu.VMEM((MAXLEN, EMB), jnp.float32),
            pltpu.VMEM((2, HALF), jnp.int32),
            pltpu.VMEM((MAXLEN, EMB), jnp.float32),
            pltpu.SemaphoreType.DMA,
        ],
    )(idx, token_table, pos_table)
    return out.reshape(BATCH, MAXLEN, EMB)


# speedup vs baseline: 1.2101x; 1.2101x over previous
import jax
import jax.numpy as jnp
from jax import lax
from jax.experimental import pallas as pl
from jax.experimental.pallas import tpu as pltpu
from jax.experimental.pallas import tpu_sc as plsc

MAXLEN = 200
EMB = 64
BATCH = 4096

NC = 2
NS = 16
NW = NC * NS

B_PER_W = BATCH // NW
HALF = 100
NBUF = 4


def _emb_body(idx_hbm, tok_hbm, pos_hbm, out_hbm,
              pos_v, idx_all, r0, r1, r2, r3,
              g0, g1, g2, g3, o0, o1, o2, o3):
    rows = (r0, r1, r2, r3)
    gsem = (g0, g1, g2, g3)
    osem = (o0, o1, o2, o3)
    wid = lax.axis_index("s") * NC + lax.axis_index("c")
    out_base = wid * B_PER_W * MAXLEN

    pltpu.sync_copy(pos_hbm, pos_v)
    pltpu.sync_copy(idx_hbm.at[pl.ds(wid * 2 * B_PER_W, 2 * B_PER_W)], idx_all)

    def fire_gather(g, b):
        pltpu.async_copy(tok_hbm.at[idx_all.at[2 * g]],
                         rows[b].at[pl.ds(0, HALF)], gsem[b])
        pltpu.async_copy(tok_hbm.at[idx_all.at[2 * g + 1]],
                         rows[b].at[pl.ds(HALF, HALF)], gsem[b])

    def drain_gather(b):
        pltpu.make_async_copy(tok_hbm.at[pl.ds(0, MAXLEN)], rows[b],
                              gsem[b]).wait()

    def drain_write(b):
        pltpu.make_async_copy(rows[b], out_hbm.at[pl.ds(0, MAXLEN)],
                              osem[b]).wait()

    fire_gather(0, 0)
    fire_gather(1, 1)

    def ring_body(p, carry):
        for b in range(NBUF):
            g = NBUF * p + b
            nxt = (b + 2) % NBUF

            @pl.when(g + 2 < B_PER_W)
            def _prefetch():
                @pl.when(g >= 2)
                def _recycle():
                    drain_write(nxt)
                fire_gather(g + 2, nxt)

            drain_gather(b)

            def add_body(r, c):
                for cc in range(EMB // 16):
                    sl = pl.ds(cc * 16, 16)
                    rows[b][r, sl] = rows[b][r, sl] + pos_v[r, sl]
                return c

            lax.fori_loop(0, MAXLEN, add_body, 0)

            pltpu.async_copy(
                rows[b], out_hbm.at[pl.ds(out_base + g * MAXLEN, MAXLEN)],
                osem[b])
        return carry

    lax.fori_loop(0, B_PER_W // NBUF, ring_body, 0)
    for b in range(NBUF):
        drain_write(b)


@jax.jit
def kernel(x, token_table, pos_table):
    idx = x.astype(jnp.int32).reshape(BATCH * MAXLEN // HALF, HALF)
    mesh = plsc.VectorSubcoreMesh(core_axis_name="c", subcore_axis_name="s")
    out = pl.kernel(
        _emb_body,
        mesh=mesh,
        compiler_params=pltpu.CompilerParams(use_tc_tiling_on_sc=False),
        out_type=jax.ShapeDtypeStruct((BATCH * MAXLEN, EMB), jnp.float32),
        scratch_types=(
            [pltpu.VMEM((MAXLEN, EMB), jnp.float32),
             pltpu.VMEM((2 * B_PER_W, HALF), jnp.int32)]
            + [pltpu.VMEM((MAXLEN, EMB), jnp.float32)] * NBUF
            + [pltpu.SemaphoreType.DMA] * (2 * NBUF)
        ),
    )(idx, token_table, pos_table)
    return out.reshape(BATCH, MAXLEN, EMB)


# speedup vs baseline: 1.2109x; 1.0007x over previous
import jax
import jax.numpy as jnp
from jax import lax
from jax.experimental import pallas as pl
from jax.experimental.pallas import tpu as pltpu
from jax.experimental.pallas import tpu_sc as plsc

MAXLEN = 200
EMB = 64
BATCH = 4096

NC = 2
NS = 16
NW = NC * NS

B_PER_W = BATCH // NW
HALF = 100
NBUF = 4


def _emb_body(idx_hbm, tok_hbm, pos_hbm, out_hbm,
              pos_v, idx_all, r0, r1, r2, r3,
              g0, g1, g2, g3, o0, o1, o2, o3):
    rows = (r0, r1, r2, r3)
    gsem = (g0, g1, g2, g3)
    osem = (o0, o1, o2, o3)
    wid = lax.axis_index("s") * NC + lax.axis_index("c")
    b_base = wid * B_PER_W

    pltpu.sync_copy(pos_hbm, pos_v)
    pltpu.sync_copy(idx_hbm.at[pl.ds(2 * b_base, 2 * B_PER_W)], idx_all)

    def fire_gather(g, b):
        for h in range(MAXLEN // HALF):
            pltpu.async_copy(tok_hbm.at[idx_all.at[2 * g + h]],
                             rows[b].at[pl.ds(h * HALF, HALF)], gsem[b])

    def drain_gather(b):
        pltpu.make_async_copy(tok_hbm.at[pl.ds(0, MAXLEN)], rows[b],
                              gsem[b]).wait()

    def drain_write(b):
        pltpu.make_async_copy(rows[b], out_hbm.at[0], osem[b]).wait()

    fire_gather(0, 0)
    fire_gather(1, 1)

    def ring_body(p, carry):
        for b in range(NBUF):
            g = NBUF * p + b
            nxt = (b + 2) % NBUF

            @pl.when(g + 2 < B_PER_W)
            def _prefetch():
                @pl.when(g >= 2)
                def _recycle():
                    drain_write(nxt)
                fire_gather(g + 2, nxt)

            drain_gather(b)

            def add_body(r, c):
                for cc in range(EMB // 16):
                    sl = pl.ds(cc * 16, 16)
                    rows[b][r, sl] = rows[b][r, sl] + pos_v[r, sl]
                return c

            lax.fori_loop(0, MAXLEN, add_body, 0)

            pltpu.async_copy(rows[b], out_hbm.at[b_base + g], osem[b])
        return carry

    lax.fori_loop(0, B_PER_W // NBUF, ring_body, 0)
    for b in range(NBUF):
        drain_write(b)


@jax.jit
def kernel(x, token_table, pos_table):
    idx = x.astype(jnp.int32).reshape(BATCH * MAXLEN // HALF, HALF)
    mesh = plsc.VectorSubcoreMesh(core_axis_name="c", subcore_axis_name="s")
    return pl.kernel(
        _emb_body,
        mesh=mesh,
        compiler_params=pltpu.CompilerParams(use_tc_tiling_on_sc=False),
        out_type=jax.ShapeDtypeStruct((BATCH, MAXLEN, EMB), jnp.float32),
        scratch_types=(
            [pltpu.VMEM((MAXLEN, EMB), jnp.float32),
             pltpu.VMEM((2 * B_PER_W, HALF), jnp.int32)]
            + [pltpu.VMEM((MAXLEN, EMB), jnp.float32)] * NBUF
            + [pltpu.SemaphoreType.DMA] * (2 * NBUF)
        ),
    )(idx, token_table, pos_table)
